# initial kernel scaffold (unmeasured)
import functools

import jax
import jax.numpy as jnp
from jax import lax
from jax.experimental import pallas as pl
from jax.experimental.pallas import tpu as pltpu

N_DEV = 32
B, SQ, SKV = 2, 128, 128
HQ_LOC, DH = 4, 64
D_MODEL = 512
CHUNK = HQ_LOC * DH
ROWS = B * SQ
MASKS = (1, 2, 4, 8, 16)


def _body(x_ref, wq_ref, kt_ref, vt_ref, wo_ref, out_ref,
          c0, c1, c2, c3, c4, send_sems, recv_sems):
    comms = (c0, c1, c2, c3, c4)
    me = lax.axis_index("i")

    barrier = pltpu.get_barrier_semaphore()
    for m in MASKS:
        pl.semaphore_signal(barrier, inc=1, device_id=(me ^ m,),
                            device_id_type=pl.DeviceIdType.MESH)
    pl.semaphore_wait(barrier, len(MASKS))

    for b in range(B):
        xb = x_ref[b]
        q = lax.dot(xb, wq_ref[...], preferred_element_type=jnp.float32)
        qb = q.astype(jnp.bfloat16)
        acc = jnp.zeros((SQ, D_MODEL), jnp.float32)
        for h in range(HQ_LOC):
            qh = qb[:, h * DH:(h + 1) * DH]
            kh = kt_ref[b, h]
            s = lax.dot_general(qh, kh, (((1,), (1,)), ((), ())),
                                preferred_element_type=jnp.float32)
            s = s * 0.125
            s = s - jnp.max(s, axis=-1, keepdims=True)
            e = jnp.exp(s)
            w = e / jnp.sum(e, axis=-1, keepdims=True)
            ctx = lax.dot(w.astype(jnp.bfloat16), vt_ref[b, h],
                          preferred_element_type=jnp.float32)
            acc = acc + lax.dot(ctx.astype(jnp.bfloat16),
                                wo_ref[h * DH:(h + 1) * DH, :],
                                preferred_element_type=jnp.float32)
        out_ref[pl.ds(b * SQ, SQ), :] = acc

    start = me * 0
    rows = ROWS
    for r, m in enumerate(MASKS):
        partner = me ^ m
        half = rows // 2
        bit = (me & m) != 0
        send_start = jnp.where(bit, start, start + half)
        keep_start = jnp.where(bit, start + half, start)
        rdma = pltpu.make_async_remote_copy(
            src_ref=out_ref.at[pl.ds(pl.multiple_of(send_start, 8), half), :],
            dst_ref=comms[r],
            send_sem=send_sems.at[r],
            recv_sem=recv_sems.at[r],
            device_id=(partner,),
            device_id_type=pl.DeviceIdType.MESH,
        )
        rdma.start()
        rdma.wait()
        ks = pl.multiple_of(keep_start, 8)
        out_ref[pl.ds(ks, half), :] = out_ref[pl.ds(ks, half), :] + comms[r][...]
        start = keep_start
        rows = half

    for r2, m in enumerate(reversed(MASKS)):
        partner = me ^ m
        bit = (me & m) != 0
        sl = pl.ds(pl.multiple_of(start, 8), rows)
        rdma = pltpu.make_async_remote_copy(
            src_ref=out_ref.at[sl, :],
            dst_ref=out_ref.at[sl, :],
            send_sem=send_sems.at[5 + r2],
            recv_sem=recv_sems.at[5 + r2],
            device_id=(partner,),
            device_id_type=pl.DeviceIdType.MESH,
        )
        rdma.start()
        rdma.wait()
        start = jnp.where(bit, start - rows, start)
        rows = rows * 2

    @functools.partial(pl.run_scoped, exit_sem=pltpu.SemaphoreType.REGULAR)
    def _(exit_sem):
        for m in MASKS:
            pl.semaphore_signal(exit_sem, inc=1, device_id=(me ^ m,),
                                device_id_type=pl.DeviceIdType.MESH)
        pl.semaphore_wait(exit_sem, len(MASKS))


def kernel(x, Wq, K_ext, V_ext, Wo):
    me = lax.axis_index("i")
    wq_loc = lax.dynamic_slice(Wq, (0, me * CHUNK), (D_MODEL, CHUNK))
    wo_loc = lax.dynamic_slice(Wo, (me * CHUNK, 0), (CHUNK, D_MODEL))
    xb = x.astype(jnp.bfloat16)
    kt = jnp.transpose(K_ext, (0, 2, 1, 3)).astype(jnp.bfloat16)
    vt = jnp.transpose(V_ext, (0, 2, 1, 3)).astype(jnp.bfloat16)

    out2d = pl.pallas_call(
        _body,
        out_shape=jax.ShapeDtypeStruct((ROWS, D_MODEL), jnp.float32),
        in_specs=[pl.BlockSpec(memory_space=pltpu.VMEM)] * 5,
        out_specs=pl.BlockSpec(memory_space=pltpu.VMEM),
        scratch_shapes=[
            pltpu.VMEM((128, D_MODEL), jnp.float32),
            pltpu.VMEM((64, D_MODEL), jnp.float32),
            pltpu.VMEM((32, D_MODEL), jnp.float32),
            pltpu.VMEM((16, D_MODEL), jnp.float32),
            pltpu.VMEM((8, D_MODEL), jnp.float32),
            pltpu.SemaphoreType.DMA((10,)),
            pltpu.SemaphoreType.DMA((10,)),
        ],
        compiler_params=pltpu.CompilerParams(collective_id=0),
    )(xb, wq_loc.astype(jnp.bfloat16), kt, vt, wo_loc.astype(jnp.bfloat16))
    return out2d.reshape(B, SQ, D_MODEL)


# baseline (device time: 38561 ns/iter reference)
import jax
import jax.numpy as jnp
from jax import lax
from jax.experimental import pallas as pl
from jax.experimental.pallas import tpu as pltpu

N_DEV = 32
B, SQ, SKV = 2, 128, 128
HQ_LOC, DH = 4, 64
D_MODEL = 512
CHUNK = HQ_LOC * DH
ROWS = B * SQ
MASKS = (1, 2, 4, 8, 16)


def _body(x_ref, wq_ref, kt_ref, vt_ref, wo_ref, out_ref,
          c0, c1, c2, c3, c4, send_sems, recv_sems):
    comms = (c0, c1, c2, c3, c4)
    me = lax.axis_index("i")

    barrier = pltpu.get_barrier_semaphore()
    for m in MASKS:
        pl.semaphore_signal(barrier, inc=1, device_id=(me ^ m,),
                            device_id_type=pl.DeviceIdType.MESH)
    pl.semaphore_wait(barrier, len(MASKS))

    for b in range(B):
        xb = x_ref[b]
        q = lax.dot(xb, wq_ref[...], preferred_element_type=jnp.float32)
        qb = q.astype(jnp.bfloat16)
        acc = jnp.zeros((SQ, D_MODEL), jnp.float32)
        for h in range(HQ_LOC):
            qh = qb[:, h * DH:(h + 1) * DH]
            kh = kt_ref[b, h]
            s = lax.dot_general(qh, kh, (((1,), (1,)), ((), ())),
                                preferred_element_type=jnp.float32)
            s = s * 0.125
            s = s - jnp.max(s, axis=-1, keepdims=True)
            e = jnp.exp(s)
            w = e / jnp.sum(e, axis=-1, keepdims=True)
            ctx = lax.dot(w.astype(jnp.bfloat16), vt_ref[b, h],
                          preferred_element_type=jnp.float32)
            acc = acc + lax.dot(ctx.astype(jnp.bfloat16),
                                wo_ref[h * DH:(h + 1) * DH, :],
                                preferred_element_type=jnp.float32)
        out_ref[pl.ds(b * SQ, SQ), :] = acc.astype(jnp.bfloat16)

    pending = []

    start = me * 0
    rows = ROWS
    for r, m in enumerate(MASKS):
        partner = me ^ m
        half = rows // 2
        bit = (me & m) != 0
        send_start = jnp.where(bit, start, start + half)
        keep_start = jnp.where(bit, start + half, start)
        rdma = pltpu.make_async_remote_copy(
            src_ref=out_ref.at[pl.ds(pl.multiple_of(send_start, 8), half), :],
            dst_ref=comms[r],
            send_sem=send_sems.at[r],
            recv_sem=recv_sems.at[r],
            device_id=(partner,),
            device_id_type=pl.DeviceIdType.MESH,
        )
        rdma.start()
        rdma.wait_recv()
        pending.append(rdma)
        ks = pl.multiple_of(keep_start, 8)
        acc = out_ref[pl.ds(ks, half), :].astype(jnp.float32) \
            + comms[r][...].astype(jnp.float32)
        out_ref[pl.ds(ks, half), :] = acc.astype(jnp.bfloat16)
        start = keep_start
        rows = half

    for r2, m in enumerate(reversed(MASKS)):
        partner = me ^ m
        bit = (me & m) != 0
        sl = pl.ds(pl.multiple_of(start, 8), rows)
        rdma = pltpu.make_async_remote_copy(
            src_ref=out_ref.at[sl, :],
            dst_ref=out_ref.at[sl, :],
            send_sem=send_sems.at[5 + r2],
            recv_sem=recv_sems.at[5 + r2],
            device_id=(partner,),
            device_id_type=pl.DeviceIdType.MESH,
        )
        rdma.start()
        rdma.wait_recv()
        pending.append(rdma)
        start = jnp.where(bit, start - rows, start)
        rows = rows * 2

    for rdma in pending:
        rdma.wait_send()


def kernel(x, Wq, K_ext, V_ext, Wo):
    me = lax.axis_index("i")
    wq_loc = lax.dynamic_slice(Wq, (0, me * CHUNK), (D_MODEL, CHUNK))
    wo_loc = lax.dynamic_slice(Wo, (me * CHUNK, 0), (CHUNK, D_MODEL))
    xb = x.astype(jnp.bfloat16)
    kt = jnp.transpose(K_ext, (0, 2, 1, 3)).astype(jnp.bfloat16)
    vt = jnp.transpose(V_ext, (0, 2, 1, 3)).astype(jnp.bfloat16)

    out2d = pl.pallas_call(
        _body,
        out_shape=jax.ShapeDtypeStruct((ROWS, D_MODEL), jnp.bfloat16),
        in_specs=[pl.BlockSpec(memory_space=pltpu.VMEM)] * 5,
        out_specs=pl.BlockSpec(memory_space=pltpu.VMEM),
        scratch_shapes=[
            pltpu.VMEM((128, D_MODEL), jnp.bfloat16),
            pltpu.VMEM((64, D_MODEL), jnp.bfloat16),
            pltpu.VMEM((32, D_MODEL), jnp.bfloat16),
            pltpu.VMEM((16, D_MODEL), jnp.bfloat16),
            pltpu.VMEM((8, D_MODEL), jnp.bfloat16),
            pltpu.SemaphoreType.DMA((10,)),
            pltpu.SemaphoreType.DMA((10,)),
        ],
        compiler_params=pltpu.CompilerParams(collective_id=0),
    )(xb, wq_loc.astype(jnp.bfloat16), kt, vt, wo_loc.astype(jnp.bfloat16))
    return out2d.astype(jnp.float32).reshape(B, SQ, D_MODEL)


# device time: 30018 ns/iter; 1.2846x vs baseline; 1.2846x over previous
import jax
import jax.numpy as jnp
from jax import lax
from jax.experimental import pallas as pl
from jax.experimental.pallas import tpu as pltpu

N_DEV = 32
B, SQ, SKV = 2, 128, 128
HQ_LOC, DH = 4, 64
D_MODEL = 512
CHUNK = HQ_LOC * DH
ROWS = B * SQ
SEG = ROWS // N_DEV


def _body(x_ref, wq_ref, kt_ref, vt_ref, wo_ref, out_ref, comm_ref,
          rs_send, rs_recv, ag_send, ag_recv):
    me = lax.axis_index("i")
    my_lo = pl.multiple_of(me * SEG, SEG)

    barrier = pltpu.get_barrier_semaphore()
    for k in range(1, N_DEV):
        pl.semaphore_signal(barrier, inc=1, device_id=((me + k) % N_DEV,),
                            device_id_type=pl.DeviceIdType.MESH)
    pl.semaphore_wait(barrier, N_DEV - 1)

    def compute_batch(b):
        xb = x_ref[b]
        q = lax.dot(xb, wq_ref[...], preferred_element_type=jnp.float32)
        qb = q.astype(jnp.bfloat16)
        acc = jnp.zeros((SQ, D_MODEL), jnp.float32)
        for h in range(HQ_LOC):
            qh = qb[:, h * DH:(h + 1) * DH]
            kh = kt_ref[b, h]
            s = lax.dot_general(qh, kh, (((1,), (1,)), ((), ())),
                                preferred_element_type=jnp.float32)
            s = s * 0.125
            s = s - jnp.max(s, axis=-1, keepdims=True)
            e = jnp.exp(s)
            w = e / jnp.sum(e, axis=-1, keepdims=True)
            ctx = lax.dot(w.astype(jnp.bfloat16), vt_ref[b, h],
                          preferred_element_type=jnp.float32)
            acc = acc + lax.dot(ctx.astype(jnp.bfloat16),
                                wo_ref[h * DH:(h + 1) * DH, :],
                                preferred_element_type=jnp.float32)
        out_ref[pl.ds(b * SQ, SQ), :] = acc.astype(jnp.bfloat16)

    def rs_send_chunk(c):
        @pl.when(me != c)
        def _():
            rdma = pltpu.make_async_remote_copy(
                src_ref=out_ref.at[pl.ds(c * SEG, SEG), :],
                dst_ref=comm_ref.at[pl.ds(my_lo, SEG), :],
                send_sem=rs_send.at[c],
                recv_sem=rs_recv.at[me],
                device_id=(c,),
                device_id_type=pl.DeviceIdType.MESH,
            )
            rdma.start()

    compute_batch(0)
    for c in range(N_DEV // 2):
        rs_send_chunk(c)
    compute_batch(1)
    for c in range(N_DEV // 2, N_DEV):
        rs_send_chunk(c)

    comm_ref[pl.ds(my_lo, SEG), :] = out_ref[pl.ds(my_lo, SEG), :]

    for s in range(N_DEV):
        @pl.when(me != s)
        def _(s=s):
            d = pltpu.make_async_remote_copy(
                src_ref=comm_ref.at[pl.ds(s * SEG, SEG), :],
                dst_ref=comm_ref.at[pl.ds(s * SEG, SEG), :],
                send_sem=rs_send.at[s],
                recv_sem=rs_recv.at[s],
                device_id=(s,),
                device_id_type=pl.DeviceIdType.MESH,
            )
            d.wait_recv()

    red = jnp.zeros((SEG, D_MODEL), jnp.float32)
    for s in range(N_DEV):
        red = red + comm_ref[pl.ds(s * SEG, SEG), :].astype(jnp.float32)
    out_ref[pl.ds(my_lo, SEG), :] = red.astype(jnp.bfloat16)

    for d_ in range(N_DEV):
        @pl.when(me != d_)
        def _(d_=d_):
            rdma = pltpu.make_async_remote_copy(
                src_ref=out_ref.at[pl.ds(my_lo, SEG), :],
                dst_ref=out_ref.at[pl.ds(my_lo, SEG), :],
                send_sem=ag_send.at[d_],
                recv_sem=ag_recv.at[me],
                device_id=(d_,),
                device_id_type=pl.DeviceIdType.MESH,
            )
            rdma.start()
    for s in range(N_DEV):
        @pl.when(me != s)
        def _(s=s):
            d = pltpu.make_async_remote_copy(
                src_ref=out_ref.at[pl.ds(s * SEG, SEG), :],
                dst_ref=out_ref.at[pl.ds(s * SEG, SEG), :],
                send_sem=ag_send.at[s],
                recv_sem=ag_recv.at[s],
                device_id=(s,),
                device_id_type=pl.DeviceIdType.MESH,
            )
            d.wait_recv()

    for c in range(N_DEV):
        @pl.when(me != c)
        def _(c=c):
            drs = pltpu.make_async_remote_copy(
                src_ref=out_ref.at[pl.ds(c * SEG, SEG), :],
                dst_ref=comm_ref.at[pl.ds(c * SEG, SEG), :],
                send_sem=rs_send.at[c],
                recv_sem=rs_recv.at[c],
                device_id=(c,),
                device_id_type=pl.DeviceIdType.MESH,
            )
            drs.wait_send()
            dag = pltpu.make_async_remote_copy(
                src_ref=out_ref.at[pl.ds(my_lo, SEG), :],
                dst_ref=out_ref.at[pl.ds(my_lo, SEG), :],
                send_sem=ag_send.at[c],
                recv_sem=ag_recv.at[c],
                device_id=(c,),
                device_id_type=pl.DeviceIdType.MESH,
            )
            dag.wait_send()


def kernel(x, Wq, K_ext, V_ext, Wo):
    me = lax.axis_index("i")
    wq_loc = lax.dynamic_slice(Wq, (0, me * CHUNK), (D_MODEL, CHUNK))
    wo_loc = lax.dynamic_slice(Wo, (me * CHUNK, 0), (CHUNK, D_MODEL))
    xb = x.astype(jnp.bfloat16)
    kt = jnp.transpose(K_ext, (0, 2, 1, 3)).astype(jnp.bfloat16)
    vt = jnp.transpose(V_ext, (0, 2, 1, 3)).astype(jnp.bfloat16)

    out2d = pl.pallas_call(
        _body,
        out_shape=jax.ShapeDtypeStruct((ROWS, D_MODEL), jnp.bfloat16),
        in_specs=[pl.BlockSpec(memory_space=pltpu.VMEM)] * 5,
        out_specs=pl.BlockSpec(memory_space=pltpu.VMEM),
        scratch_shapes=[
            pltpu.VMEM((ROWS, D_MODEL), jnp.bfloat16),
            pltpu.SemaphoreType.DMA((N_DEV,)),
            pltpu.SemaphoreType.DMA((N_DEV,)),
            pltpu.SemaphoreType.DMA((N_DEV,)),
            pltpu.SemaphoreType.DMA((N_DEV,)),
        ],
        compiler_params=pltpu.CompilerParams(collective_id=0),
    )(xb, wq_loc.astype(jnp.bfloat16), kt, vt, wo_loc.astype(jnp.bfloat16))
    return out2d.astype(jnp.float32).reshape(B, SQ, D_MODEL)
